# baseline (speedup 1.0000x reference)
import functools

import jax
import jax.numpy as jnp
from jax import lax
from jax.experimental import pallas as pl
from jax.experimental.pallas import tpu as pltpu
from jax.experimental.pallas import tpu_sc as plsc

NN = 50000
NE = 800000
NG = 512
EMB = 32
HID = 64
OUT = 16
NREL = 3
F1 = 48
NB = 1000
GRID = NN // NB


def _emb_body(sid_ref, cid_ref, pid_ref, se_ref, ce_ref, pe_ref, out_ref):
    oh_s = (sid_ref[...] == lax.broadcasted_iota(jnp.int32, (NB, 64), 1)).astype(jnp.float32)
    oh_c = (cid_ref[...] == lax.broadcasted_iota(jnp.int32, (NB, 64), 1)).astype(jnp.float32)
    oh_p = (pid_ref[...] == lax.broadcasted_iota(jnp.int32, (NB, 256), 1)).astype(jnp.float32)
    out_ref[...] = oh_s @ se_ref[...] + oh_c @ ce_ref[...] + oh_p @ pe_ref[...]


def _embed(sid, cid, pid, se48, ce48, pe48):
    return pl.pallas_call(
        _emb_body,
        grid=(GRID,),
        in_specs=[
            pl.BlockSpec((NB, 1), lambda i: (i, 0)),
            pl.BlockSpec((NB, 1), lambda i: (i, 0)),
            pl.BlockSpec((NB, 1), lambda i: (i, 0)),
            pl.BlockSpec((64, F1), lambda i: (0, 0)),
            pl.BlockSpec((64, F1), lambda i: (0, 0)),
            pl.BlockSpec((256, F1), lambda i: (0, 0)),
        ],
        out_specs=pl.BlockSpec((NB, F1), lambda i: (i, 0)),
        out_shape=jax.ShapeDtypeStruct((NN, F1), jnp.float32),
    )(sid, cid, pid, se48, ce48, pe48)


def _t1_body(xp_ref, agg_ref, w1r_ref, w1root_ref, b1_ref, h1_ref, rinv_ref):
    xp = xp_ref[...]
    x = xp[:, :EMB]
    agg = agg_ref[...]
    acc = x @ w1root_ref[...] + b1_ref[...]
    rinvs = []
    for r in range(NREL):
        s = agg[:, r * F1:r * F1 + EMB]
        c = agg[:, r * F1 + EMB:r * F1 + EMB + 1]
        ri = 1.0 / jnp.maximum(c, 1.0)
        rinvs.append(ri)
        acc += (s * ri) @ w1r_ref[r * EMB:(r + 1) * EMB, :]
    h1_ref[...] = jnp.maximum(acc, 0.0)
    rinv_ref[...] = jnp.concatenate(rinvs + [jnp.zeros((NB, 5), jnp.float32)], axis=1)


def _layer1(x48, agg1, w1r, w1root, b1):
    return pl.pallas_call(
        _t1_body,
        grid=(GRID,),
        in_specs=[
            pl.BlockSpec((NB, F1), lambda i: (i, 0)),
            pl.BlockSpec((NB, NREL * F1), lambda i: (i, 0)),
            pl.BlockSpec((NREL * EMB, HID), lambda i: (0, 0)),
            pl.BlockSpec((EMB, HID), lambda i: (0, 0)),
            pl.BlockSpec((1, HID), lambda i: (0, 0)),
        ],
        out_specs=[
            pl.BlockSpec((NB, HID), lambda i: (i, 0)),
            pl.BlockSpec((NB, 8), lambda i: (i, 0)),
        ],
        out_shape=[
            jax.ShapeDtypeStruct((NN, HID), jnp.float32),
            jax.ShapeDtypeStruct((NN, 8), jnp.float32),
        ],
    )(x48, agg1, w1r, w1root, b1)


def _t2_body(h1_ref, agg_ref, rinv_ref, batch_ref, w2r_ref, w2root_ref,
             b2_ref, wlin_ref, blin_ref, out_ref, pool_ref, gcnt_ref):
    i = pl.program_id(0)
    h1 = h1_ref[...]
    agg = agg_ref[...]
    rinv = rinv_ref[...]
    acc = h1 @ w2root_ref[...] + b2_ref[...]
    for r in range(NREL):
        acc += (agg[:, r * HID:(r + 1) * HID] * rinv[:, r:r + 1]) @ w2r_ref[r * HID:(r + 1) * HID, :]
    h2 = jnp.maximum(acc, 0.0)
    M = (batch_ref[...] == lax.broadcasted_iota(jnp.int32, (NB, NG), 1)).astype(jnp.float32)
    P = lax.dot_general(M, h2, (((0,), (0,)), ((), ())),
                        preferred_element_type=jnp.float32)
    g = lax.dot_general(M, jnp.ones((NB, 1), jnp.float32), (((0,), (0,)), ((), ())),
                        preferred_element_type=jnp.float32)

    @pl.when(i == 0)
    def _():
        pool_ref[...] = jnp.zeros_like(pool_ref)
        gcnt_ref[...] = jnp.zeros_like(gcnt_ref)

    pool_ref[...] += P
    gcnt_ref[...] += g

    @pl.when(i == GRID - 1)
    def _():
        out_ref[...] = (pool_ref[...] / jnp.maximum(gcnt_ref[...], 1.0)) @ wlin_ref[...] + blin_ref[...]


def _layer2_pool(h1, agg2, rinv, batch2d, w2r, w2root, b2, wlin, blin):
    return pl.pallas_call(
        _t2_body,
        grid=(GRID,),
        in_specs=[
            pl.BlockSpec((NB, HID), lambda i: (i, 0)),
            pl.BlockSpec((NB, NREL * HID), lambda i: (i, 0)),
            pl.BlockSpec((NB, 8), lambda i: (i, 0)),
            pl.BlockSpec((NB, 1), lambda i: (i, 0)),
            pl.BlockSpec((NREL * HID, HID), lambda i: (0, 0)),
            pl.BlockSpec((HID, HID), lambda i: (0, 0)),
            pl.BlockSpec((1, HID), lambda i: (0, 0)),
            pl.BlockSpec((HID, OUT), lambda i: (0, 0)),
            pl.BlockSpec((1, OUT), lambda i: (0, 0)),
        ],
        out_specs=pl.BlockSpec((NG, OUT), lambda i: (0, 0)),
        out_shape=jax.ShapeDtypeStruct((NG, OUT), jnp.float32),
        scratch_shapes=[
            pltpu.VMEM((NG, HID), jnp.float32),
            pltpu.VMEM((NG, 1), jnp.float32),
        ],
    )(h1, agg2, rinv, batch2d, w2r, w2root, b2, wlin, blin)


NSUB = 16
NEP = 819200
EPS = NEP // NSUB
GB = 256


def _make_agg(F, CE, D, R, NRPC, ZR):
    NBLK = CE // GB
    CHN = EPS // CE
    ZROWS = R * 3 // NSUB
    NZ = ZROWS // ZR
    mesh = plsc.VectorSubcoreMesh(core_axis_name="c", subcore_axis_name="s")

    def body(src_hbm, dst_hbm, typ_hbm, x_hbm, z_hbm, out_hbm,
             agg_sp, src_b, dst_b, typ_b, *slots):
        gis = slots[0:D]
        sis = slots[D:2 * D]
        rows = slots[2 * D:3 * D]
        gsem = slots[3 * D:4 * D]
        ssem = slots[4 * D:5 * D]
        c = lax.axis_index("c")
        s = lax.axis_index("s")
        estart = s * EPS
        zoff = s * ZROWS
        for ri in range(NRPC):
            base = (c * NRPC + ri) * R

            def zero_body(z, _):
                pltpu.sync_copy(z_hbm, agg_sp.at[pl.ds(zoff + z * ZR, ZR)])
                return 0
            lax.fori_loop(0, NZ, zero_body, 0)
            plsc.subcore_barrier()

            def chunk_body(ch, _):
                off = estart + ch * CE
                pltpu.sync_copy(src_hbm.at[pl.ds(off, CE)], src_b)
                pltpu.sync_copy(dst_hbm.at[pl.ds(off, CE)], dst_b)
                pltpu.sync_copy(typ_hbm.at[pl.ds(off, CE)], typ_b)
                gdescs = [None] * D
                sdescs = [None] * D
                for b in range(NBLK):
                    p = b % D
                    if b >= D:
                        sdescs[p].wait()
                    for j in range(GB // 16):
                        o = b * GB + j * 16
                        dv = dst_b[pl.ds(o, 16)]
                        sv = src_b[pl.ds(o, 16)]
                        tv = typ_b[pl.ds(o, 16)]
                        m = (dv >= base) & (dv < base + R)
                        li = jnp.where(m, (dv - base) * 3 + tv, R * 3)
                        gis[p][pl.ds(j * 16, 16)] = sv
                        sis[p][pl.ds(j * 16, 16)] = li
                    gdescs[p] = pltpu.async_copy(x_hbm.at[gis[p]],
                                                 rows[p], gsem[p])
                    if b > 0:
                        q = (b - 1) % D
                        gdescs[q].wait()
                        sdescs[q] = pltpu.async_copy(
                            rows[q], agg_sp.at[sis[q]], ssem[q], add=True)
                q = (NBLK - 1) % D
                gdescs[q].wait()
                sdescs[q] = pltpu.async_copy(rows[q], agg_sp.at[sis[q]],
                                             ssem[q], add=True)
                for p in range(D):
                    if sdescs[p] is not None:
                        sdescs[p].wait()
                return 0
            lax.fori_loop(0, CHN, chunk_body, 0)
            plsc.subcore_barrier()
            pltpu.sync_copy(agg_sp.at[pl.ds(zoff, ZROWS)],
                            out_hbm.at[pl.ds(base * 3 + zoff, ZROWS)])
            plsc.subcore_barrier()

    kern = pl.kernel(
        body,
        out_type=jax.ShapeDtypeStruct((2 * NRPC * R * 3, F), jnp.float32),
        mesh=mesh,
        compiler_params=pltpu.CompilerParams(use_tc_tiling_on_sc=False),
        scratch_types=[
            pltpu.VMEM_SHARED((R * 3 + 16, F), jnp.float32),
            pltpu.VMEM((CE,), jnp.int32),
            pltpu.VMEM((CE,), jnp.int32),
            pltpu.VMEM((CE,), jnp.int32),
        ] + [pltpu.VMEM((GB,), jnp.int32) for _ in range(2 * D)]
          + [pltpu.VMEM((GB, F), jnp.float32) for _ in range(D)]
          + [pltpu.SemaphoreType.DMA for _ in range(2 * D)],
    )
    return kern


_agg_l1 = _make_agg(F1, 2048, 3, 8448, 3, 528)
_agg_l2 = _make_agg(HID, 2048, 2, 6272, 4, 588)


def kernel(sid, cid, pid, edge_index, edge_type, batch,
           shape_emb, col_emb, pos_emb,
           W1_rel, W1_root, b1, W2_rel, W2_root, b2, W_lin, b_lin):
    sid2 = sid.reshape(NN, 1).astype(jnp.int32)
    cid2 = cid.reshape(NN, 1).astype(jnp.int32)
    pid2 = pid.reshape(NN, 1).astype(jnp.int32)
    batch2 = batch.reshape(NN, 1).astype(jnp.int32)

    def pad48(t, ones_col):
        p = jnp.zeros((t.shape[0], F1), jnp.float32).at[:, :EMB].set(t)
        if ones_col:
            p = p.at[:, EMB].set(1.0)
        return p

    se48 = pad48(shape_emb, True)
    ce48 = pad48(col_emb, False)
    pe48 = pad48(pos_emb, False)

    x48 = _embed(sid2, cid2, pid2, se48, ce48, pe48)

    npad = NEP - NE
    src = jnp.concatenate([edge_index[0].astype(jnp.int32),
                           jnp.zeros((npad,), jnp.int32)])
    dst = jnp.concatenate([edge_index[1].astype(jnp.int32),
                           jnp.full((npad,), 1 << 30, jnp.int32)])
    et = jnp.concatenate([edge_type.astype(jnp.int32),
                          jnp.zeros((npad,), jnp.int32)])
    z48 = jnp.zeros((528, F1), jnp.float32)
    z64 = jnp.zeros((588, HID), jnp.float32)

    agg1 = _agg_l1(src, dst, et, x48, z48)[:NN * NREL].reshape(NN, NREL * F1)

    w1r = W1_rel.reshape(NREL * EMB, HID)
    h1, rinv = _layer1(x48, agg1, w1r, W1_root, b1.reshape(1, HID))

    agg2 = _agg_l2(src, dst, et, h1, z64)[:NN * NREL].reshape(NN, NREL * HID)

    w2r = W2_rel.reshape(NREL * HID, HID)
    out = _layer2_pool(h1, agg2, rinv, batch2, w2r, W2_root,
                       b2.reshape(1, HID), W_lin.reshape(HID, OUT),
                       b_lin.reshape(1, OUT))
    return out


# speedup vs baseline: 2.0200x; 2.0200x over previous
import functools

import jax
import jax.numpy as jnp
from jax import lax
from jax.experimental import pallas as pl
from jax.experimental.pallas import tpu as pltpu
from jax.experimental.pallas import tpu_sc as plsc

NN = 50000
NE = 800000
NG = 512
EMB = 32
HID = 64
OUT = 16
NREL = 3
F1 = 48
NB = 1000
GRID = NN // NB


def _emb_body(sid_ref, cid_ref, pid_ref, se_ref, ce_ref, pe_ref, out_ref):
    oh_s = (sid_ref[...] == lax.broadcasted_iota(jnp.int32, (NB, 64), 1)).astype(jnp.float32)
    oh_c = (cid_ref[...] == lax.broadcasted_iota(jnp.int32, (NB, 64), 1)).astype(jnp.float32)
    oh_p = (pid_ref[...] == lax.broadcasted_iota(jnp.int32, (NB, 256), 1)).astype(jnp.float32)
    out_ref[...] = oh_s @ se_ref[...] + oh_c @ ce_ref[...] + oh_p @ pe_ref[...]


def _embed(sid, cid, pid, se48, ce48, pe48):
    return pl.pallas_call(
        _emb_body,
        grid=(GRID,),
        in_specs=[
            pl.BlockSpec((NB, 1), lambda i: (i, 0)),
            pl.BlockSpec((NB, 1), lambda i: (i, 0)),
            pl.BlockSpec((NB, 1), lambda i: (i, 0)),
            pl.BlockSpec((64, F1), lambda i: (0, 0)),
            pl.BlockSpec((64, F1), lambda i: (0, 0)),
            pl.BlockSpec((256, F1), lambda i: (0, 0)),
        ],
        out_specs=pl.BlockSpec((NB, F1), lambda i: (i, 0)),
        out_shape=jax.ShapeDtypeStruct((NN, F1), jnp.float32),
    )(sid, cid, pid, se48, ce48, pe48)


def _t1_body(xp_ref, agg_ref, w1r_ref, w1root_ref, b1_ref, h1_ref, rinv_ref):
    xp = xp_ref[...]
    x = xp[:, :EMB]
    agg = agg_ref[...]
    acc = x @ w1root_ref[...] + b1_ref[...]
    rinvs = []
    for r in range(NREL):
        s = agg[:, r * F1:r * F1 + EMB]
        c = agg[:, r * F1 + EMB:r * F1 + EMB + 1]
        ri = 1.0 / jnp.maximum(c, 1.0)
        rinvs.append(ri)
        acc += (s * ri) @ w1r_ref[r * EMB:(r + 1) * EMB, :]
    h1_ref[...] = jnp.maximum(acc, 0.0)
    rinv_ref[...] = jnp.concatenate(rinvs + [jnp.zeros((NB, 5), jnp.float32)], axis=1)


def _layer1(x48, agg1, w1r, w1root, b1):
    return pl.pallas_call(
        _t1_body,
        grid=(GRID,),
        in_specs=[
            pl.BlockSpec((NB, F1), lambda i: (i, 0)),
            pl.BlockSpec((NB, NREL * F1), lambda i: (i, 0)),
            pl.BlockSpec((NREL * EMB, HID), lambda i: (0, 0)),
            pl.BlockSpec((EMB, HID), lambda i: (0, 0)),
            pl.BlockSpec((1, HID), lambda i: (0, 0)),
        ],
        out_specs=[
            pl.BlockSpec((NB, HID), lambda i: (i, 0)),
            pl.BlockSpec((NB, 8), lambda i: (i, 0)),
        ],
        out_shape=[
            jax.ShapeDtypeStruct((NN, HID), jnp.float32),
            jax.ShapeDtypeStruct((NN, 8), jnp.float32),
        ],
    )(x48, agg1, w1r, w1root, b1)


def _t2_body(h1_ref, agg_ref, rinv_ref, batch_ref, w2r_ref, w2root_ref,
             b2_ref, wlin_ref, blin_ref, out_ref, pool_ref, gcnt_ref):
    i = pl.program_id(0)
    h1 = h1_ref[...]
    agg = agg_ref[...]
    rinv = rinv_ref[...]
    acc = h1 @ w2root_ref[...] + b2_ref[...]
    for r in range(NREL):
        acc += (agg[:, r * HID:(r + 1) * HID] * rinv[:, r:r + 1]) @ w2r_ref[r * HID:(r + 1) * HID, :]
    h2 = jnp.maximum(acc, 0.0)
    M = (batch_ref[...] == lax.broadcasted_iota(jnp.int32, (NB, NG), 1)).astype(jnp.float32)
    P = lax.dot_general(M, h2, (((0,), (0,)), ((), ())),
                        preferred_element_type=jnp.float32)
    g = lax.dot_general(M, jnp.ones((NB, 1), jnp.float32), (((0,), (0,)), ((), ())),
                        preferred_element_type=jnp.float32)

    @pl.when(i == 0)
    def _():
        pool_ref[...] = jnp.zeros_like(pool_ref)
        gcnt_ref[...] = jnp.zeros_like(gcnt_ref)

    pool_ref[...] += P
    gcnt_ref[...] += g

    @pl.when(i == GRID - 1)
    def _():
        out_ref[...] = (pool_ref[...] / jnp.maximum(gcnt_ref[...], 1.0)) @ wlin_ref[...] + blin_ref[...]


def _layer2_pool(h1, agg2, rinv, batch2d, w2r, w2root, b2, wlin, blin):
    return pl.pallas_call(
        _t2_body,
        grid=(GRID,),
        in_specs=[
            pl.BlockSpec((NB, HID), lambda i: (i, 0)),
            pl.BlockSpec((NB, NREL * HID), lambda i: (i, 0)),
            pl.BlockSpec((NB, 8), lambda i: (i, 0)),
            pl.BlockSpec((NB, 1), lambda i: (i, 0)),
            pl.BlockSpec((NREL * HID, HID), lambda i: (0, 0)),
            pl.BlockSpec((HID, HID), lambda i: (0, 0)),
            pl.BlockSpec((1, HID), lambda i: (0, 0)),
            pl.BlockSpec((HID, OUT), lambda i: (0, 0)),
            pl.BlockSpec((1, OUT), lambda i: (0, 0)),
        ],
        out_specs=pl.BlockSpec((NG, OUT), lambda i: (0, 0)),
        out_shape=jax.ShapeDtypeStruct((NG, OUT), jnp.float32),
        scratch_shapes=[
            pltpu.VMEM((NG, HID), jnp.float32),
            pltpu.VMEM((NG, 1), jnp.float32),
        ],
    )(h1, agg2, rinv, batch2d, w2r, w2root, b2, wlin, blin)


NSUB = 16
NEP = 819200
EPS = NEP // NSUB
GB = 128


def _make_agg(F, CE, R, NRPC, ZR):
    MAXB = CE // GB
    CHN = EPS // CE
    ZROWS = R * 3 // NSUB
    NZ = ZROWS // ZR
    mesh = plsc.VectorSubcoreMesh(core_axis_name="c", subcore_axis_name="s")

    def body(src_hbm, dst_hbm, typ_hbm, x_hbm, z_hbm, lane_hbm, out_hbm,
             agg_sp, src_b, dst_b, typ_b, gi_buf, si_buf, lane_b,
             sc0, sc1, r0, r1, gs0, gs1, ss0, ss1):
        rows = [r0, r1]
        gsem = [gs0, gs1]
        ssem = [ss0, ss1]
        scur = [sc0, sc1]
        c = lax.axis_index("c")
        s = lax.axis_index("s")
        estart = s * EPS
        zoff = s * ZROWS
        pltpu.sync_copy(lane_hbm, lane_b)
        lane = lane_b[pl.ds(0, 16)]
        stages = []
        for k in (2, 4, 8, 16):
            j = k // 2
            while j >= 1:
                pidx = lane ^ j
                keep_small = ((lane & j) == 0) == ((lane & k) == 0)
                stages.append((pidx, keep_small))
                j //= 2
        for ri in range(NRPC):
            base = (c * NRPC + ri) * R

            def zero_body(z, _):
                pltpu.sync_copy(z_hbm, agg_sp.at[pl.ds(zoff + z * ZR, ZR)])
                return 0
            lax.fori_loop(0, NZ, zero_body, 0)
            plsc.subcore_barrier()

            def chunk_body(ch, _):
                off = estart + ch * CE
                pltpu.sync_copy(src_hbm.at[pl.ds(off, CE)], src_b)
                pltpu.sync_copy(dst_hbm.at[pl.ds(off, CE)], dst_b)
                pltpu.sync_copy(typ_hbm.at[pl.ds(off, CE)], typ_b)

                def grp(g, ptr):
                    dv = dst_b[pl.ds(g * 16, 16)]
                    sv = src_b[pl.ds(g * 16, 16)]
                    tv = typ_b[pl.ds(g * 16, 16)]
                    m = (dv >= base) & (dv < base + R)
                    li = jnp.where(m, (dv - base) * 3 + tv, R * 3)
                    key = jnp.where(m, lane, lane + 16)
                    val = lane
                    for pidx, ks in stages:
                        pkey = key[pidx]
                        pval = val[pidx]
                        nkey = jnp.where(ks, jnp.minimum(key, pkey),
                                         jnp.maximum(key, pkey))
                        val = jnp.where(nkey != key, pval, val)
                        key = nkey
                    gi_buf[pl.ds(ptr, 16)] = sv[val]
                    si_buf[pl.ds(ptr, 16)] = li[val]
                    return ptr + plsc.all_reduce_population_count(m)[0]
                ptr = lax.fori_loop(0, CE // 16, grp, 0)

                dummy_g = jnp.zeros((16,), jnp.int32)
                dummy_s = jnp.full((16,), R * 3, jnp.int32)
                for j in range(GB // 16 + 1):
                    gi_buf[pl.ds(ptr + j * 16, 16)] = dummy_g
                    si_buf[pl.ds(ptr + j * 16, 16)] = dummy_s
                nblk = (ptr + GB - 1) // GB

                gd = [None] * MAXB
                sd = [None] * MAXB
                for b in range(MAXB):
                    pp = b % 2
                    if b >= 2:
                        @pl.when(b - 2 < nblk)
                        def _(b=b):
                            sd[b - 2].wait()

                    @pl.when(b < nblk)
                    def _(b=b, pp=pp):
                        gd[b] = pltpu.async_copy(
                            x_hbm.at[gi_buf.at[pl.ds(b * GB, GB)]],
                            rows[pp], gsem[pp])
                    if b >= 1:
                        q = (b - 1) % 2

                        @pl.when(b - 1 < nblk)
                        def _(b=b, q=q):
                            gd[b - 1].wait()
                            for t in range(GB // 16):
                                scur[q][pl.ds(t * 16, 16)] = \
                                    si_buf[pl.ds((b - 1) * GB + t * 16, 16)]
                            sd[b - 1] = pltpu.async_copy(
                                rows[q], agg_sp.at[scur[q]], ssem[q], add=True)
                bl = MAXB - 1
                q = bl % 2

                @pl.when(bl < nblk)
                def _():
                    gd[bl].wait()
                    for t in range(GB // 16):
                        scur[q][pl.ds(t * 16, 16)] = \
                            si_buf[pl.ds(bl * GB + t * 16, 16)]
                    sd[bl] = pltpu.async_copy(rows[q], agg_sp.at[scur[q]],
                                              ssem[q], add=True)
                for t in (MAXB - 2, MAXB - 1):
                    @pl.when(t < nblk)
                    def _(t=t):
                        sd[t].wait()
                return 0
            lax.fori_loop(0, CHN, chunk_body, 0)
            plsc.subcore_barrier()
            pltpu.sync_copy(agg_sp.at[pl.ds(zoff, ZROWS)],
                            out_hbm.at[pl.ds(base * 3 + zoff, ZROWS)])
            plsc.subcore_barrier()

    kern = pl.kernel(
        body,
        out_type=jax.ShapeDtypeStruct((2 * NRPC * R * 3, F), jnp.float32),
        mesh=mesh,
        compiler_params=pltpu.CompilerParams(use_tc_tiling_on_sc=False, needs_layout_passes=False),
        scratch_types=[
            pltpu.VMEM_SHARED((R * 3 + 16, F), jnp.float32),
            pltpu.VMEM((CE,), jnp.int32),
            pltpu.VMEM((CE,), jnp.int32),
            pltpu.VMEM((CE,), jnp.int32),
            pltpu.VMEM((CE + 2 * GB,), jnp.int32),
            pltpu.VMEM((CE + 2 * GB,), jnp.int32),
            pltpu.VMEM((16,), jnp.int32),
            pltpu.VMEM((GB,), jnp.int32),
            pltpu.VMEM((GB,), jnp.int32),
            pltpu.VMEM((GB, F), jnp.float32),
            pltpu.VMEM((GB, F), jnp.float32),
            pltpu.SemaphoreType.DMA,
            pltpu.SemaphoreType.DMA,
            pltpu.SemaphoreType.DMA,
            pltpu.SemaphoreType.DMA,
        ],
    )
    return kern


_agg_l1 = _make_agg(F1, 2048, 8448, 3, 528)
_agg_l2 = _make_agg(HID, 2048, 8448, 3, 528)


def kernel(sid, cid, pid, edge_index, edge_type, batch,
           shape_emb, col_emb, pos_emb,
           W1_rel, W1_root, b1, W2_rel, W2_root, b2, W_lin, b_lin):
    sid2 = sid.reshape(NN, 1).astype(jnp.int32)
    cid2 = cid.reshape(NN, 1).astype(jnp.int32)
    pid2 = pid.reshape(NN, 1).astype(jnp.int32)
    batch2 = batch.reshape(NN, 1).astype(jnp.int32)

    def pad48(t, ones_col):
        p = jnp.zeros((t.shape[0], F1), jnp.float32).at[:, :EMB].set(t)
        if ones_col:
            p = p.at[:, EMB].set(1.0)
        return p

    se48 = pad48(shape_emb, True)
    ce48 = pad48(col_emb, False)
    pe48 = pad48(pos_emb, False)

    x48 = _embed(sid2, cid2, pid2, se48, ce48, pe48)

    npad = NEP - NE
    src = jnp.concatenate([edge_index[0].astype(jnp.int32),
                           jnp.zeros((npad,), jnp.int32)])
    dst = jnp.concatenate([edge_index[1].astype(jnp.int32),
                           jnp.full((npad,), 1 << 30, jnp.int32)])
    et = jnp.concatenate([edge_type.astype(jnp.int32),
                          jnp.zeros((npad,), jnp.int32)])
    z48 = jnp.zeros((528, F1), jnp.float32)
    z64 = jnp.zeros((528, HID), jnp.float32)
    lanes = jnp.arange(16, dtype=jnp.int32)

    agg1 = _agg_l1(src, dst, et, x48, z48, lanes)[:NN * NREL].reshape(NN, NREL * F1)

    w1r = W1_rel.reshape(NREL * EMB, HID)
    h1, rinv = _layer1(x48, agg1, w1r, W1_root, b1.reshape(1, HID))

    agg2 = _agg_l2(src, dst, et, h1, z64, lanes)[:NN * NREL].reshape(NN, NREL * HID)

    w2r = W2_rel.reshape(NREL * HID, HID)
    out = _layer2_pool(h1, agg2, rinv, batch2, w2r, W2_root,
                       b2.reshape(1, HID), W_lin.reshape(HID, OUT),
                       b_lin.reshape(1, OUT))
    return out
